# baseline (speedup 1.0000x reference)
import functools

import jax
import jax.numpy as jnp
from jax import lax
from jax.experimental import pallas as pl
from jax.experimental.pallas import tpu as pltpu, tpu_sc as plsc

NUM_CORES = 2
NUM_SUBCORES = 16
NW = NUM_CORES * NUM_SUBCORES
LANES = 16
CHUNK = 128
BLK = 32768
RPW = 11264
RS = NW * RPW
CN = 512


def _tc_body(dim, wb_ref, wrows_ref, ut_ref, it_ref, pu_ref, pi_ref):
    dn = (((1,), (0,)), ((), ()))
    ru = jax.lax.dot_general(wrows_ref[0:1, :], ut_ref[...], dn,
                             preferred_element_type=jnp.float32)
    ri = jax.lax.dot_general(wrows_ref[1:2, :], it_ref[...], dn,
                             preferred_element_type=jnp.float32)
    pu_ref[:] = ru.reshape(ru.shape[1]) + wb_ref[2 * dim]
    pi_ref[:] = ri.reshape(ri.shape[1])


def _make_tc_call(n_rows, dim):
    nr = n_rows - RS
    off = RS // BLK
    grid = (pl.cdiv(nr, BLK),)
    return pl.pallas_call(
        functools.partial(_tc_body, dim),
        grid=grid,
        in_specs=[
            pl.BlockSpec(memory_space=pltpu.SMEM),
            pl.BlockSpec((2, dim), lambda g: (0, 0)),
            pl.BlockSpec((dim, BLK), lambda g: (0, g + off)),
            pl.BlockSpec((dim, BLK), lambda g: (0, g + off)),
        ],
        out_specs=[
            pl.BlockSpec((BLK,), lambda g: (g,)),
            pl.BlockSpec((BLK,), lambda g: (g,)),
        ],
        out_shape=[
            jax.ShapeDtypeStruct((nr,), jnp.float32),
            jax.ShapeDtypeStruct((nr,), jnp.float32),
        ],
    )


def _sc_matvec_body(dim, ut, it, wb, pu, pi,
                    tvu0, tvi0, tvu1, tvi1, pou, poi, wvm,
                    sem0, sem1):
    wid = lax.axis_index("s") * NUM_CORES + lax.axis_index("c")
    r0 = wid * RPW
    pltpu.sync_copy(wb, wvm)
    wvecs = [wvm[pl.ds(k * LANES, LANES)] for k in range(5)]
    wall = [wvecs[k][l] for k in range(5) for l in range(LANES)]
    wu = wall[:dim]
    wi = wall[dim:2 * dim]
    bias = wall[2 * dim]

    bufs = [(tvu0, tvi0, sem0), (tvu1, tvi1, sem1)]
    nchunk = RPW // CN

    def start(c, p):
        tu, ti, sem = bufs[p]
        col = pl.ds(r0 + c * CN, CN)
        pltpu.async_copy(ut.at[:, col], tu, sem)
        pltpu.async_copy(it.at[:, col], ti, sem)

    def drain(p):
        tu, ti, sem = bufs[p]
        pltpu.make_async_copy(ut.at[:, pl.ds(0, CN)], tu, sem).wait()
        pltpu.make_async_copy(it.at[:, pl.ds(0, CN)], ti, sem).wait()

    def make_grp(tu, ti):
        def grp(g):
            sl = pl.ds(g * LANES, LANES)
            au = tu[0, sl] * wu[0]
            ai = ti[0, sl] * wi[0]
            for j in range(1, dim):
                au += tu[j, sl] * wu[j]
                ai += ti[j, sl] * wi[j]
            pou[sl] = au + bias
            poi[sl] = ai
        return grp

    start(0, 0)
    start(1, 1)

    def pair(k, carry):
        for p in range(2):
            c = 2 * k + p
            tu, ti, _sem = bufs[p]
            drain(p)
            plsc.parallel_loop(0, CN // LANES, unroll=1)(make_grp(tu, ti))
            ocol = pl.ds(r0 + c * CN, CN)
            pltpu.sync_copy(pou, pu.at[ocol])
            pltpu.sync_copy(poi, pi.at[ocol])
            start(c + 2, p)
        return carry

    lax.fori_loop(0, nchunk // 2, pair, None)
    drain(0)
    drain(1)


def _make_sc_matvec_call(dim):
    return pl.kernel(
        functools.partial(_sc_matvec_body, dim),
        out_type=[
            jax.ShapeDtypeStruct((RS,), jnp.float32),
            jax.ShapeDtypeStruct((RS,), jnp.float32),
        ],
        mesh=plsc.VectorSubcoreMesh(
            core_axis_name="c", subcore_axis_name="s",
            num_cores=NUM_CORES, num_subcores=NUM_SUBCORES),
        compiler_params=pltpu.CompilerParams(
            needs_layout_passes=False, use_tc_tiling_on_sc=True),
        scratch_types=[
            pltpu.VMEM((32, CN), jnp.float32),
            pltpu.VMEM((32, CN), jnp.float32),
            pltpu.VMEM((32, CN), jnp.float32),
            pltpu.VMEM((32, CN), jnp.float32),
            pltpu.VMEM((CN,), jnp.float32),
            pltpu.VMEM((CN,), jnp.float32),
            pltpu.VMEM((5 * LANES,), jnp.float32),
            pltpu.SemaphoreType.DMA,
            pltpu.SemaphoreType.DMA,
        ],
    )


def _sc_body(bpw, user_ids, item_ids, pu_s, pi_s, pu_t, pi_t, out,
             uidx, iidx, us_ix, ut_ix, is_ix, it_ix,
             gu_s, gu_t, gi_s, gi_t, sem):
    wid = lax.axis_index("s") * NUM_CORES + lax.axis_index("c")
    base = wid * bpw

    pltpu.sync_copy(user_ids.at[pl.ds(base, bpw)], uidx)
    pltpu.sync_copy(item_ids.at[pl.ds(base, bpw)], iidx)

    def split(g, _):
        sl = pl.ds(g * LANES, LANES)
        u = uidx[sl]
        i = iidx[sl]
        us_ix[sl] = u % RS
        is_ix[sl] = i % RS
        ut_ix[sl] = jnp.where(u >= RS, u - RS, u)
        it_ix[sl] = jnp.where(i >= RS, i - RS, i)
        return _

    lax.fori_loop(0, bpw // LANES, split, None)

    copies = []
    for c in range(bpw // CHUNK):
        sl = pl.ds(c * CHUNK, CHUNK)
        copies.append(pltpu.async_copy(pu_s.at[us_ix.at[sl]], gu_s.at[sl], sem))
        copies.append(pltpu.async_copy(pu_t.at[ut_ix.at[sl]], gu_t.at[sl], sem))
        copies.append(pltpu.async_copy(pi_s.at[is_ix.at[sl]], gi_s.at[sl], sem))
        copies.append(pltpu.async_copy(pi_t.at[it_ix.at[sl]], gi_t.at[sl], sem))
    for cp in copies:
        cp.wait()

    for g in range(bpw // LANES):
        sl = pl.ds(g * LANES, LANES)
        vu = jnp.where(uidx[sl] < RS, gu_s[sl], gu_t[sl])
        vi = jnp.where(iidx[sl] < RS, gi_s[sl], gi_t[sl])
        s = vu + vi
        gu_s[sl] = 1.0 / (1.0 + jnp.exp(-s))

    pltpu.sync_copy(gu_s, out.at[pl.ds(base, bpw)])


def _make_sc_call(batch):
    bpw = batch // NW
    return pl.kernel(
        functools.partial(_sc_body, bpw),
        out_type=jax.ShapeDtypeStruct((batch,), jnp.float32),
        mesh=plsc.VectorSubcoreMesh(
            core_axis_name="c", subcore_axis_name="s",
            num_cores=NUM_CORES, num_subcores=NUM_SUBCORES),
        compiler_params=pltpu.CompilerParams(
            needs_layout_passes=False, use_tc_tiling_on_sc=False),
        scratch_types=[
            pltpu.VMEM((bpw,), jnp.int32),
            pltpu.VMEM((bpw,), jnp.int32),
            pltpu.VMEM((bpw,), jnp.int32),
            pltpu.VMEM((bpw,), jnp.int32),
            pltpu.VMEM((bpw,), jnp.int32),
            pltpu.VMEM((bpw,), jnp.int32),
            pltpu.VMEM((bpw,), jnp.float32),
            pltpu.VMEM((bpw,), jnp.float32),
            pltpu.VMEM((bpw,), jnp.float32),
            pltpu.VMEM((bpw,), jnp.float32),
            pltpu.SemaphoreType.DMA,
        ],
    )


@jax.jit
def kernel(user_ids, item_ids, user_table, item_table, W, b):
    batch = user_ids.shape[0]
    n_rows, dim = user_table.shape
    wb = jnp.concatenate(
        [W.reshape(-1), jnp.full((LANES,), b[0], jnp.float32)])
    wrows = W.reshape(2, dim)
    ut_t = user_table.T
    it_t = item_table.T
    pu_s, pi_s = _make_sc_matvec_call(dim)(ut_t, it_t, wb)
    pu_t, pi_t = _make_tc_call(n_rows, dim)(wb, wrows, ut_t, it_t)
    out = _make_sc_call(batch)(
        user_ids.astype(jnp.int32), item_ids.astype(jnp.int32),
        pu_s, pi_s, pu_t, pi_t)
    return out.reshape(batch, 1)


# speedup vs baseline: 1.0003x; 1.0003x over previous
import functools

import jax
import jax.numpy as jnp
from jax import lax
from jax.experimental import pallas as pl
from jax.experimental.pallas import tpu as pltpu, tpu_sc as plsc

NUM_CORES = 2
NUM_SUBCORES = 16
NW = NUM_CORES * NUM_SUBCORES
LANES = 16
CHUNK = 128
BLK = 32768
RPW = 9216
RS = NW * RPW
CN = 512


def _tc_body(dim, wb_ref, wrows_ref, ut_ref, it_ref, pu_ref, pi_ref):
    dn = (((1,), (0,)), ((), ()))
    ru = jax.lax.dot_general(wrows_ref[0:1, :], ut_ref[...], dn,
                             preferred_element_type=jnp.float32)
    ri = jax.lax.dot_general(wrows_ref[1:2, :], it_ref[...], dn,
                             preferred_element_type=jnp.float32)
    pu_ref[:] = ru.reshape(ru.shape[1]) + wb_ref[2 * dim]
    pi_ref[:] = ri.reshape(ri.shape[1])


def _make_tc_call(n_rows, dim):
    nr = n_rows - RS
    off = RS // BLK
    grid = (pl.cdiv(nr, BLK),)
    return pl.pallas_call(
        functools.partial(_tc_body, dim),
        grid=grid,
        in_specs=[
            pl.BlockSpec(memory_space=pltpu.SMEM),
            pl.BlockSpec((2, dim), lambda g: (0, 0)),
            pl.BlockSpec((dim, BLK), lambda g: (0, g + off)),
            pl.BlockSpec((dim, BLK), lambda g: (0, g + off)),
        ],
        out_specs=[
            pl.BlockSpec((BLK,), lambda g: (g,)),
            pl.BlockSpec((BLK,), lambda g: (g,)),
        ],
        out_shape=[
            jax.ShapeDtypeStruct((nr,), jnp.float32),
            jax.ShapeDtypeStruct((nr,), jnp.float32),
        ],
    )


def _sc_matvec_body(dim, ut, it, wb, pu, pi,
                    tvu0, tvi0, tvu1, tvi1, pou, poi, wvm,
                    sem0, sem1):
    wid = lax.axis_index("s") * NUM_CORES + lax.axis_index("c")
    r0 = wid * RPW
    pltpu.sync_copy(wb, wvm)
    wvecs = [wvm[pl.ds(k * LANES, LANES)] for k in range(5)]
    wall = [wvecs[k][l] for k in range(5) for l in range(LANES)]
    wu = wall[:dim]
    wi = wall[dim:2 * dim]
    bias = wall[2 * dim]

    bufs = [(tvu0, tvi0, sem0), (tvu1, tvi1, sem1)]
    nchunk = RPW // CN

    def start(c, p):
        tu, ti, sem = bufs[p]
        col = pl.ds(r0 + c * CN, CN)
        pltpu.async_copy(ut.at[:, col], tu, sem)
        pltpu.async_copy(it.at[:, col], ti, sem)

    def drain(p):
        tu, ti, sem = bufs[p]
        pltpu.make_async_copy(ut.at[:, pl.ds(0, CN)], tu, sem).wait()
        pltpu.make_async_copy(it.at[:, pl.ds(0, CN)], ti, sem).wait()

    def make_grp(tu, ti):
        def grp(g):
            sl = pl.ds(g * LANES, LANES)
            au = tu[0, sl] * wu[0]
            ai = ti[0, sl] * wi[0]
            for j in range(1, dim):
                au += tu[j, sl] * wu[j]
                ai += ti[j, sl] * wi[j]
            pou[sl] = au + bias
            poi[sl] = ai
        return grp

    start(0, 0)
    start(1, 1)

    def pair(k, carry):
        for p in range(2):
            c = 2 * k + p
            tu, ti, _sem = bufs[p]
            drain(p)
            plsc.parallel_loop(0, CN // LANES, unroll=1)(make_grp(tu, ti))
            ocol = pl.ds(r0 + c * CN, CN)
            pltpu.sync_copy(pou, pu.at[ocol])
            pltpu.sync_copy(poi, pi.at[ocol])
            start(c + 2, p)
        return carry

    lax.fori_loop(0, nchunk // 2, pair, None)
    drain(0)
    drain(1)


def _make_sc_matvec_call(dim):
    return pl.kernel(
        functools.partial(_sc_matvec_body, dim),
        out_type=[
            jax.ShapeDtypeStruct((RS,), jnp.float32),
            jax.ShapeDtypeStruct((RS,), jnp.float32),
        ],
        mesh=plsc.VectorSubcoreMesh(
            core_axis_name="c", subcore_axis_name="s",
            num_cores=NUM_CORES, num_subcores=NUM_SUBCORES),
        compiler_params=pltpu.CompilerParams(
            needs_layout_passes=False, use_tc_tiling_on_sc=True),
        scratch_types=[
            pltpu.VMEM((32, CN), jnp.float32),
            pltpu.VMEM((32, CN), jnp.float32),
            pltpu.VMEM((32, CN), jnp.float32),
            pltpu.VMEM((32, CN), jnp.float32),
            pltpu.VMEM((CN,), jnp.float32),
            pltpu.VMEM((CN,), jnp.float32),
            pltpu.VMEM((5 * LANES,), jnp.float32),
            pltpu.SemaphoreType.DMA,
            pltpu.SemaphoreType.DMA,
        ],
    )


def _sc_body(bpw, user_ids, item_ids, pu_s, pi_s, pu_t, pi_t, out,
             uidx, iidx, us_ix, ut_ix, is_ix, it_ix,
             gu_s, gu_t, gi_s, gi_t, sem):
    wid = lax.axis_index("s") * NUM_CORES + lax.axis_index("c")
    base = wid * bpw

    pltpu.sync_copy(user_ids.at[pl.ds(base, bpw)], uidx)
    pltpu.sync_copy(item_ids.at[pl.ds(base, bpw)], iidx)

    def split(g, _):
        sl = pl.ds(g * LANES, LANES)
        u = uidx[sl]
        i = iidx[sl]
        us_ix[sl] = u % RS
        is_ix[sl] = i % RS
        ut_ix[sl] = jnp.where(u >= RS, u - RS, u)
        it_ix[sl] = jnp.where(i >= RS, i - RS, i)
        return _

    lax.fori_loop(0, bpw // LANES, split, None)

    copies = []
    for c in range(bpw // CHUNK):
        sl = pl.ds(c * CHUNK, CHUNK)
        copies.append(pltpu.async_copy(pu_s.at[us_ix.at[sl]], gu_s.at[sl], sem))
        copies.append(pltpu.async_copy(pu_t.at[ut_ix.at[sl]], gu_t.at[sl], sem))
        copies.append(pltpu.async_copy(pi_s.at[is_ix.at[sl]], gi_s.at[sl], sem))
        copies.append(pltpu.async_copy(pi_t.at[it_ix.at[sl]], gi_t.at[sl], sem))
    for cp in copies:
        cp.wait()

    for g in range(bpw // LANES):
        sl = pl.ds(g * LANES, LANES)
        vu = jnp.where(uidx[sl] < RS, gu_s[sl], gu_t[sl])
        vi = jnp.where(iidx[sl] < RS, gi_s[sl], gi_t[sl])
        s = vu + vi
        gu_s[sl] = 1.0 / (1.0 + jnp.exp(-s))

    pltpu.sync_copy(gu_s, out.at[pl.ds(base, bpw)])


def _make_sc_call(batch):
    bpw = batch // NW
    return pl.kernel(
        functools.partial(_sc_body, bpw),
        out_type=jax.ShapeDtypeStruct((batch,), jnp.float32),
        mesh=plsc.VectorSubcoreMesh(
            core_axis_name="c", subcore_axis_name="s",
            num_cores=NUM_CORES, num_subcores=NUM_SUBCORES),
        compiler_params=pltpu.CompilerParams(
            needs_layout_passes=False, use_tc_tiling_on_sc=False),
        scratch_types=[
            pltpu.VMEM((bpw,), jnp.int32),
            pltpu.VMEM((bpw,), jnp.int32),
            pltpu.VMEM((bpw,), jnp.int32),
            pltpu.VMEM((bpw,), jnp.int32),
            pltpu.VMEM((bpw,), jnp.int32),
            pltpu.VMEM((bpw,), jnp.int32),
            pltpu.VMEM((bpw,), jnp.float32),
            pltpu.VMEM((bpw,), jnp.float32),
            pltpu.VMEM((bpw,), jnp.float32),
            pltpu.VMEM((bpw,), jnp.float32),
            pltpu.SemaphoreType.DMA,
        ],
    )


@jax.jit
def kernel(user_ids, item_ids, user_table, item_table, W, b):
    batch = user_ids.shape[0]
    n_rows, dim = user_table.shape
    wb = jnp.concatenate(
        [W.reshape(-1), jnp.full((LANES,), b[0], jnp.float32)])
    wrows = W.reshape(2, dim)
    ut_t = user_table.T
    it_t = item_table.T
    pu_s, pi_s = _make_sc_matvec_call(dim)(ut_t, it_t, wb)
    pu_t, pi_t = _make_tc_call(n_rows, dim)(wb, wrows, ut_t, it_t)
    out = _make_sc_call(batch)(
        user_ids.astype(jnp.int32), item_ids.astype(jnp.int32),
        pu_s, pi_s, pu_t, pi_t)
    return out.reshape(batch, 1)


# speedup vs baseline: 1.0762x; 1.0759x over previous
import functools

import jax
import jax.numpy as jnp
from jax import lax
from jax.experimental import pallas as pl
from jax.experimental.pallas import tpu as pltpu, tpu_sc as plsc

NUM_CORES = 2
NUM_SUBCORES = 16
NW = NUM_CORES * NUM_SUBCORES
LANES = 16
CHUNK = 128
BLK = 40960


def _tc_body(dim, wb_ref, wrows_ref, ut_ref, it_ref, pu_ref, pi_ref):
    dn = (((1,), (0,)), ((), ()))
    ru = jax.lax.dot_general(wrows_ref[0:1, :], ut_ref[...], dn,
                             preferred_element_type=jnp.float32)
    ri = jax.lax.dot_general(wrows_ref[1:2, :], it_ref[...], dn,
                             preferred_element_type=jnp.float32)
    pu_ref[:] = ru.reshape(ru.shape[1]) + wb_ref[2 * dim]
    pi_ref[:] = ri.reshape(ri.shape[1])


def _make_tc_call(n_rows, dim):
    grid = (pl.cdiv(n_rows, BLK),)
    return pl.pallas_call(
        functools.partial(_tc_body, dim),
        grid=grid,
        in_specs=[
            pl.BlockSpec(memory_space=pltpu.SMEM),
            pl.BlockSpec((2, dim), lambda g: (0, 0)),
            pl.BlockSpec((dim, BLK), lambda g: (0, g)),
            pl.BlockSpec((dim, BLK), lambda g: (0, g)),
        ],
        out_specs=[
            pl.BlockSpec((BLK,), lambda g: (g,)),
            pl.BlockSpec((BLK,), lambda g: (g,)),
        ],
        out_shape=[
            jax.ShapeDtypeStruct((n_rows,), jnp.float32),
            jax.ShapeDtypeStruct((n_rows,), jnp.float32),
        ],
    )


def _sc_body(bpw, user_ids, item_ids, pu, pi, out,
             uidx, iidx, gu, gi, sem):
    wid = lax.axis_index("s") * NUM_CORES + lax.axis_index("c")
    base = wid * bpw

    pltpu.sync_copy(user_ids.at[pl.ds(base, bpw)], uidx)
    pltpu.sync_copy(item_ids.at[pl.ds(base, bpw)], iidx)

    copies = []
    for c in range(bpw // CHUNK):
        sl = pl.ds(c * CHUNK, CHUNK)
        copies.append(pltpu.async_copy(pu.at[uidx.at[sl]], gu.at[sl], sem))
        copies.append(pltpu.async_copy(pi.at[iidx.at[sl]], gi.at[sl], sem))
    for cp in copies:
        cp.wait()

    for g in range(bpw // LANES):
        sl = pl.ds(g * LANES, LANES)
        s = gu[sl] + gi[sl]
        gu[sl] = 1.0 / (1.0 + jnp.exp(-s))

    pltpu.sync_copy(gu, out.at[pl.ds(base, bpw)])


def _make_sc_call(batch):
    bpw = batch // NW
    return pl.kernel(
        functools.partial(_sc_body, bpw),
        out_type=jax.ShapeDtypeStruct((batch,), jnp.float32),
        mesh=plsc.VectorSubcoreMesh(
            core_axis_name="c", subcore_axis_name="s",
            num_cores=NUM_CORES, num_subcores=NUM_SUBCORES),
        compiler_params=pltpu.CompilerParams(
            needs_layout_passes=False, use_tc_tiling_on_sc=False),
        scratch_types=[
            pltpu.VMEM((bpw,), jnp.int32),
            pltpu.VMEM((bpw,), jnp.int32),
            pltpu.VMEM((bpw,), jnp.float32),
            pltpu.VMEM((bpw,), jnp.float32),
            pltpu.SemaphoreType.DMA,
        ],
    )


@jax.jit
def kernel(user_ids, item_ids, user_table, item_table, W, b):
    batch = user_ids.shape[0]
    n_rows, dim = user_table.shape
    wb = jnp.concatenate(
        [W.reshape(-1), jnp.full((LANES,), b[0], jnp.float32)])
    wrows = W.reshape(2, dim)
    pu, pi = _make_tc_call(n_rows, dim)(
        wb, wrows, user_table.T, item_table.T)
    out = _make_sc_call(batch)(
        user_ids.astype(jnp.int32), item_ids.astype(jnp.int32), pu, pi)
    return out.reshape(batch, 1)


# speedup vs baseline: 1.0816x; 1.0050x over previous
import functools

import jax
import jax.numpy as jnp
from jax import lax
from jax.experimental import pallas as pl
from jax.experimental.pallas import tpu as pltpu, tpu_sc as plsc

NUM_CORES = 2
NUM_SUBCORES = 16
NW = NUM_CORES * NUM_SUBCORES
LANES = 16
CHUNK = 128
BLK = 32768


def _tc_body(dim, wb_ref, wrows_ref, ut_ref, it_ref, pu_ref, pi_ref):
    dn = (((1,), (0,)), ((), ()))
    ru = jax.lax.dot_general(wrows_ref[0:1, :], ut_ref[...], dn,
                             preferred_element_type=jnp.float32)
    ri = jax.lax.dot_general(wrows_ref[1:2, :], it_ref[...], dn,
                             preferred_element_type=jnp.float32)
    pu_ref[:] = ru.reshape(ru.shape[1]) + wb_ref[2 * dim]
    pi_ref[:] = ri.reshape(ri.shape[1])


def _make_tc_call(n_rows, dim):
    grid = (pl.cdiv(n_rows, BLK),)
    return pl.pallas_call(
        functools.partial(_tc_body, dim),
        grid=grid,
        in_specs=[
            pl.BlockSpec(memory_space=pltpu.SMEM),
            pl.BlockSpec((2, dim), lambda g: (0, 0)),
            pl.BlockSpec((dim, BLK), lambda g: (0, g)),
            pl.BlockSpec((dim, BLK), lambda g: (0, g)),
        ],
        out_specs=[
            pl.BlockSpec((BLK,), lambda g: (g,)),
            pl.BlockSpec((BLK,), lambda g: (g,)),
        ],
        out_shape=[
            jax.ShapeDtypeStruct((n_rows,), jnp.float32),
            jax.ShapeDtypeStruct((n_rows,), jnp.float32),
        ],
    )


def _sc_body(bpw, user_ids, item_ids, pu, pi, out,
             uidx, iidx, gu, gi, sem):
    wid = lax.axis_index("s") * NUM_CORES + lax.axis_index("c")
    base = wid * bpw

    pltpu.sync_copy(user_ids.at[pl.ds(base, bpw)], uidx)
    pltpu.sync_copy(item_ids.at[pl.ds(base, bpw)], iidx)

    copies = []
    for c in range(bpw // CHUNK):
        sl = pl.ds(c * CHUNK, CHUNK)
        copies.append(pltpu.async_copy(pu.at[uidx.at[sl]], gu.at[sl], sem))
        copies.append(pltpu.async_copy(pi.at[iidx.at[sl]], gi.at[sl], sem))
    for cp in copies:
        cp.wait()

    for g in range(bpw // LANES):
        sl = pl.ds(g * LANES, LANES)
        s = gu[sl] + gi[sl]
        gu[sl] = 1.0 / (1.0 + jnp.exp(-s))

    pltpu.sync_copy(gu, out.at[pl.ds(base, bpw)])


def _make_sc_call(batch):
    bpw = batch // NW
    return pl.kernel(
        functools.partial(_sc_body, bpw),
        out_type=jax.ShapeDtypeStruct((batch,), jnp.float32),
        mesh=plsc.VectorSubcoreMesh(
            core_axis_name="c", subcore_axis_name="s",
            num_cores=NUM_CORES, num_subcores=NUM_SUBCORES),
        compiler_params=pltpu.CompilerParams(
            needs_layout_passes=False, use_tc_tiling_on_sc=False),
        scratch_types=[
            pltpu.VMEM((bpw,), jnp.int32),
            pltpu.VMEM((bpw,), jnp.int32),
            pltpu.VMEM((bpw,), jnp.float32),
            pltpu.VMEM((bpw,), jnp.float32),
            pltpu.SemaphoreType.DMA,
        ],
    )


@jax.jit
def kernel(user_ids, item_ids, user_table, item_table, W, b):
    batch = user_ids.shape[0]
    n_rows, dim = user_table.shape
    wb = jnp.concatenate(
        [W.reshape(-1), jnp.full((LANES,), b[0], jnp.float32)])
    wrows = W.reshape(2, dim)
    pu, pi = _make_tc_call(n_rows, dim)(
        wb, wrows, user_table.T, item_table.T)
    out = _make_sc_call(batch)(
        user_ids.astype(jnp.int32), item_ids.astype(jnp.int32), pu, pi)
    return out.reshape(batch, 1)
